# Initial kernel scaffold; baseline (speedup 1.0000x reference)
import jax
import jax.numpy as jnp
from jax.experimental import pallas as pl


def kernel(x, edge_index, conv_w0, conv_b0, bn_g0, bn_b0, aw1_0, ab1_0, aw2_0, ab2_0, conv_w1, conv_b1, bn_g1, bn_b1, aw1_1, ab1_1, aw2_1, ab2_1, conv_w2, conv_b2, bn_g2, bn_b2, aw1_2, ab1_2, aw2_2, ab2_2, se_w1, se_b1, se_w2, se_b2, gat1_w, gat1_asrc, gat1_adst, gat1_bias, g1bn_g, g1bn_b, skip_w, skip_b, gat2_w, gat2_asrc, gat2_adst, gat2_bias, g2bn_g, g2bn_b, fc_w, fc_b):
    raise NotImplementedError("write your pallas kernel here")


# speedup vs baseline: 7.4453x; 7.4453x over previous
import jax
import jax.numpy as jnp
from jax import lax
from jax.experimental import pallas as pl

B = 128
NC = 64
NS = 256
TCH = 16
KS = (3, 5, 7)
GC = 32
H = 4
AH = 32
NCLS = 2
E0 = 1024
N = B * NC
CF = TCH * len(KS)

_R1 = 128
_G1 = N // _R1
_R2 = 512
_G2 = N // _R2
_BPG = _R2 // NC

_INTERPRET = False
_DBG = False


def _gelu(x):
    return 0.5 * x * (1.0 + lax.erf(x * 0.7071067811865476))


_PREC = lax.Precision.HIGHEST
_PDEF = lax.Precision.DEFAULT


def _dotT(a, b, prec=_PDEF):
    return lax.dot_general(a, b, (((1,), (1,)), ((), ())),
                           preferred_element_type=jnp.float32,
                           precision=prec)


def _dot(a, b, prec=_PDEF):
    return lax.dot_general(a, b, (((1,), (0,)), ((), ())),
                           preferred_element_type=jnp.float32,
                           precision=prec)


def _conv_channels(xb, w, bb, k):
    p = k // 2
    xp = jnp.pad(xb, ((0, 0), (p, p)))
    shifts = [xp[:, j:j + NS].astype(jnp.bfloat16).astype(jnp.float32)
              for j in range(k)]
    wr = w.astype(jnp.bfloat16).astype(jnp.float32)
    ys = []
    for c in range(TCH):
        acc = shifts[0] * wr[c, 0]
        for j in range(1, k):
            acc = acc + shifts[j] * wr[c, j]
        ys.append(acc + bb[0, c])
    return ys


def _conv_stats_kernel(x_ref, w0, b0, w1, b1, w2, b2, stats_ref):
    i = pl.program_id(0)

    @pl.when(i == 0)
    def _():
        stats_ref[...] = jnp.zeros_like(stats_ref)

    xb = x_ref[...]
    rows = []
    rows2 = []
    for (w_r, b_r, k) in ((w0, b0, 3), (w1, b1, 5), (w2, b2, 7)):
        ys = _conv_channels(xb, w_r[...], b_r[...], k)
        for y in ys:
            rows.append(jnp.sum(y, axis=0, keepdims=True))
            rows2.append(jnp.sum(y * y, axis=0, keepdims=True))
    stats_ref[...] += jnp.concatenate(rows + rows2, axis=0)


def _conv_apply_kernel(x_ref, stats_ref,
                       w0, b0, g0, be0, aw10, ab10, aw20, ab20,
                       w1, b1, g1, be1, aw11, ab11, aw21, ab21,
                       w2, b2, g2, be2, aw12, ab12, aw22, ab22,
                       xc_ref, colsum_ref):
    i = pl.program_id(0)

    @pl.when(i == 0)
    def _():
        colsum_ref[...] = jnp.zeros_like(colsum_ref)

    xb = x_ref[...]
    ntot = jnp.float32(N * NS)
    tot = jnp.sum(stats_ref[...], axis=1, keepdims=True)
    feats = []
    br = 0
    for (w_r, b_r, g_r, be_r, aw1_r, ab1_r, aw2_r, ab2_r, k) in (
            (w0, b0, g0, be0, aw10, ab10, aw20, ab20, 3),
            (w1, b1, g1, be1, aw11, ab11, aw21, ab21, 5),
            (w2, b2, g2, be2, aw12, ab12, aw22, ab22, 7)):
        ys = _conv_channels(xb, w_r[...], b_r[...], k)
        g_v = g_r[...]
        be_v = be_r[...]
        gl = []
        for c in range(TCH):
            m = tot[br * TCH + c, 0] / ntot
            v = tot[48 + br * TCH + c, 0] / ntot - m * m
            sc = g_v[0, c] * lax.rsqrt(v + 1e-5)
            sh = be_v[0, c] - m * sc
            gl.append(_gelu(ys[c] * sc + sh))
        gs = jnp.stack(gl, axis=1)
        R = gs.shape[0]
        aw1b = jnp.broadcast_to(aw1_r[...][None], (R, AH, TCH))
        z = lax.dot_general(aw1b, gs, (((2,), (1,)), ((0,), (0,))),
                            preferred_element_type=jnp.float32,
                            precision=_PDEF)
        zt = jnp.tanh(z + ab1_r[...][:, :, None])
        aw2b = jnp.broadcast_to(aw2_r[...][None], (R, 1, AH))
        sc2 = lax.dot_general(aw2b, zt, (((2,), (1,)), ((0,), (0,))),
                              preferred_element_type=jnp.float32,
                              precision=_PDEF)[:, 0, :]
        sc2 = sc2 + ab2_r[...][0, 0]
        mx = jnp.max(sc2, axis=1, keepdims=True)
        e = jnp.exp(sc2 - mx)
        wgt = e / jnp.sum(e, axis=1, keepdims=True)
        feat = lax.dot_general(gs, wgt[:, :, None], (((2,), (1,)), ((0,), (0,))),
                               preferred_element_type=jnp.float32,
                               precision=_PREC)[:, :, 0]
        feats.append(feat)
        br += 1
    xcb = jnp.concatenate(feats, axis=1)
    xc_ref[...] = xcb
    colsum_ref[0:1, :] += jnp.sum(xcb, axis=0, keepdims=True)


def _se_proj_kernel(xc_ref, colsum_ref, sew1, seb1, sew2, seb2,
                    g1w, skw, skb, h1_ref, xs_ref):
    m = colsum_ref[0:1, :] * (1.0 / N)
    t1 = _gelu(_dotT(m, sew1[...]) + seb1[...])
    s = jax.nn.sigmoid(_dotT(t1, sew2[...]) + seb2[...])
    xcs = xc_ref[...] * s
    h1_ref[...] = _dotT(xcs, g1w[...])
    xs_ref[...] = _dotT(xcs, skw[...]) + skb[...]


def _gat_kernel(h_ref, ei_ref, asrc_ref, adst_ref, bias_ref,
                out_ref, stats_ref):
    i = pl.program_id(0)

    @pl.when(i == 0)
    def _():
        stats_ref[...] = jnp.zeros_like(stats_ref)

    src = ei_ref[0:1, :]
    dst = ei_ref[1:2, :]
    node_iota = lax.broadcasted_iota(jnp.int32, (NC, E0), 0)
    sm = (src == node_iota).astype(jnp.float32)
    dm = (dst == node_iota).astype(jnp.float32)
    cmat = _dotT(dm, sm)
    ri = lax.broadcasted_iota(jnp.int32, (NC, NC), 0)
    ci = lax.broadcasted_iota(jnp.int32, (NC, NC), 1)
    cmat = cmat + (ri == ci).astype(jnp.float32)
    maskneg = jnp.where(cmat > 0, 0.0, -1e30).astype(jnp.float32)

    asrc = asrc_ref[...]
    adst = adst_ref[...]
    bias = bias_ref[...]
    ssum = jnp.zeros((1, GC), jnp.float32)
    ssq = jnp.zeros((1, GC), jnp.float32)
    outs = []
    for b in range(_BPG):
        hb = h_ref[b * NC:(b + 1) * NC, :]
        acc = jnp.zeros((NC, GC), jnp.float32)
        for k in range(H):
            hk = hb[:, k * GC:(k + 1) * GC]
            a_s = _dotT(asrc[k:k + 1, :], hk, prec=_PREC)
            a_d = jnp.sum(hk * adst[k:k + 1, :], axis=1, keepdims=True)
            al = a_d + a_s
            al = jnp.where(al > 0, al, 0.2 * al)
            am = jnp.max(al + maskneg, axis=1, keepdims=True)
            e = jnp.exp(al - am) * cmat
            den = jnp.sum(e, axis=1, keepdims=True) + 1e-16
            acc = acc + _dot(e / den, hk, prec=_PREC)
        ob = acc * (1.0 / H) + bias
        outs.append(ob)
        ssum = ssum + jnp.sum(ob, axis=0, keepdims=True)
        ssq = ssq + jnp.sum(ob * ob, axis=0, keepdims=True)
    out_ref[...] = jnp.concatenate(outs, axis=0)
    stats_ref[0:1, :] += ssum
    stats_ref[1:2, :] += ssq


def _bn_skip_proj_kernel(x1_ref, stats_ref, xs_ref, g_ref, b_ref, w2_ref,
                         x1f_ref, h2_ref):
    st = stats_ref[...]
    m = st[0:1, :] * (1.0 / N)
    v = st[1:2, :] * (1.0 / N) - m * m
    sc = g_ref[...] * lax.rsqrt(v + 1e-5)
    sh = b_ref[...] - m * sc
    x1f = _gelu(x1_ref[...] * sc + sh + xs_ref[...])
    x1f_ref[...] = x1f
    h2_ref[...] = _dotT(x1f, w2_ref[...])


def _final_kernel(x2_ref, stats_ref, x1f_ref, g_ref, b_ref, fcw_ref, fcb_ref,
                  out_ref):
    st = stats_ref[...]
    m = st[0:1, :] * (1.0 / N)
    v = st[1:2, :] * (1.0 / N) - m * m
    sc = g_ref[...] * lax.rsqrt(v + 1e-5)
    sh = b_ref[...] - m * sc
    y = _gelu(x2_ref[...] * sc + sh + x1f_ref[...])
    ps = [jnp.sum(y[b * NC:(b + 1) * NC, :], axis=0, keepdims=True) * (1.0 / NC)
          for b in range(_BPG)]
    pooled = jnp.concatenate(ps, axis=0)
    out_ref[...] = _dotT(pooled, fcw_ref[...]) + fcb_ref[...]


def _full(shape):
    return pl.BlockSpec(shape, lambda i: (0,) * len(shape))


def kernel(x, edge_index,
           conv_w0, conv_b0, bn_g0, bn_b0, aw1_0, ab1_0, aw2_0, ab2_0,
           conv_w1, conv_b1, bn_g1, bn_b1, aw1_1, ab1_1, aw2_1, ab2_1,
           conv_w2, conv_b2, bn_g2, bn_b2, aw1_2, ab1_2, aw2_2, ab2_2,
           se_w1, se_b1, se_w2, se_b2,
           gat1_w, gat1_asrc, gat1_adst, gat1_bias, g1bn_g, g1bn_b,
           skip_w, skip_b,
           gat2_w, gat2_asrc, gat2_adst, gat2_bias, g2bn_g, g2bn_b,
           fc_w, fc_b):
    f32 = jnp.float32
    x2d = x.reshape(N, NS)
    ei = jnp.pad(edge_index, ((0, 6), (0, 0)))
    w0 = conv_w0.reshape(TCH, 3)
    w1 = conv_w1.reshape(TCH, 5)
    w2 = conv_w2.reshape(TCH, 7)
    b0 = conv_b0.reshape(1, TCH)
    b1 = conv_b1.reshape(1, TCH)
    b2 = conv_b2.reshape(1, TCH)
    g0 = bn_g0.reshape(1, TCH); be0 = bn_b0.reshape(1, TCH)
    g1 = bn_g1.reshape(1, TCH); be1 = bn_b1.reshape(1, TCH)
    g2 = bn_g2.reshape(1, TCH); be2 = bn_b2.reshape(1, TCH)
    a10 = aw1_0.reshape(AH, TCH); a11 = aw1_1.reshape(AH, TCH)
    a12 = aw1_2.reshape(AH, TCH)
    ab10 = ab1_0.reshape(1, AH); ab11 = ab1_1.reshape(1, AH)
    ab12 = ab1_2.reshape(1, AH)
    a20 = aw2_0.reshape(1, AH); a21 = aw2_1.reshape(1, AH)
    a22 = aw2_2.reshape(1, AH)
    ab20 = ab2_0.reshape(1, 1); ab21 = ab2_1.reshape(1, 1)
    ab22 = ab2_2.reshape(1, 1)
    seb1 = se_b1.reshape(1, CF // 4)
    seb2 = se_b2.reshape(1, CF)
    g1bias = gat1_bias.reshape(1, GC)
    g2bias = gat2_bias.reshape(1, GC)
    skb = skip_b.reshape(1, GC)
    g1g = g1bn_g.reshape(1, GC); g1b = g1bn_b.reshape(1, GC)
    g2g = g2bn_g.reshape(1, GC); g2b = g2bn_b.reshape(1, GC)
    fcb = fc_b.reshape(1, NCLS)

    stats = pl.pallas_call(
        _conv_stats_kernel,
        grid=(_G1,),
        in_specs=[pl.BlockSpec((_R1, NS), lambda i: (i, 0)),
                  _full(w0.shape), _full(b0.shape),
                  _full(w1.shape), _full(b1.shape),
                  _full(w2.shape), _full(b2.shape)],
        out_specs=_full((96, NS)),
        out_shape=jax.ShapeDtypeStruct((96, NS), f32),
        interpret=_INTERPRET,
    )(x2d, w0, b0, w1, b1, w2, b2)

    xc, colsum = pl.pallas_call(
        _conv_apply_kernel,
        grid=(_G1,),
        in_specs=[pl.BlockSpec((_R1, NS), lambda i: (i, 0)), _full((96, NS)),
                  _full(w0.shape), _full(b0.shape), _full(g0.shape),
                  _full(be0.shape), _full(a10.shape), _full(ab10.shape),
                  _full(a20.shape), _full(ab20.shape),
                  _full(w1.shape), _full(b1.shape), _full(g1.shape),
                  _full(be1.shape), _full(a11.shape), _full(ab11.shape),
                  _full(a21.shape), _full(ab21.shape),
                  _full(w2.shape), _full(b2.shape), _full(g2.shape),
                  _full(be2.shape), _full(a12.shape), _full(ab12.shape),
                  _full(a22.shape), _full(ab22.shape)],
        out_specs=[pl.BlockSpec((_R1, CF), lambda i: (i, 0)),
                   _full((8, CF))],
        out_shape=[jax.ShapeDtypeStruct((N, CF), f32),
                   jax.ShapeDtypeStruct((8, CF), f32)],
        interpret=_INTERPRET,
    )(x2d, stats, w0, b0, g0, be0, a10, ab10, a20, ab20,
      w1, b1, g1, be1, a11, ab11, a21, ab21,
      w2, b2, g2, be2, a12, ab12, a22, ab22)

    h1, xs = pl.pallas_call(
        _se_proj_kernel,
        grid=(_G2,),
        in_specs=[pl.BlockSpec((_R2, CF), lambda i: (i, 0)), _full((8, CF)),
                  _full(se_w1.shape), _full(seb1.shape),
                  _full(se_w2.shape), _full(seb2.shape),
                  _full(gat1_w.shape), _full(skip_w.shape), _full(skb.shape)],
        out_specs=[pl.BlockSpec((_R2, H * GC), lambda i: (i, 0)),
                   pl.BlockSpec((_R2, GC), lambda i: (i, 0))],
        out_shape=[jax.ShapeDtypeStruct((N, H * GC), f32),
                   jax.ShapeDtypeStruct((N, GC), f32)],
        interpret=_INTERPRET,
    )(xc, colsum, se_w1, seb1, se_w2, seb2, gat1_w, skip_w, skb)

    def gat(h, asrc, adst, bias):
        return pl.pallas_call(
            _gat_kernel,
            grid=(_G2,),
            in_specs=[pl.BlockSpec((_R2, H * GC), lambda i: (i, 0)),
                      _full((8, E0)),
                      _full((H, GC)), _full((H, GC)), _full((1, GC))],
            out_specs=[pl.BlockSpec((_R2, GC), lambda i: (i, 0)),
                       _full((8, GC))],
            out_shape=[jax.ShapeDtypeStruct((N, GC), f32),
                       jax.ShapeDtypeStruct((8, GC), f32)],
            interpret=_INTERPRET,
        )(h, ei, asrc, adst, bias)

    x1raw, stats1 = gat(h1, gat1_asrc, gat1_adst, g1bias)

    x1f, h2 = pl.pallas_call(
        _bn_skip_proj_kernel,
        grid=(_G2,),
        in_specs=[pl.BlockSpec((_R2, GC), lambda i: (i, 0)), _full((8, GC)),
                  pl.BlockSpec((_R2, GC), lambda i: (i, 0)),
                  _full((1, GC)), _full((1, GC)), _full(gat2_w.shape)],
        out_specs=[pl.BlockSpec((_R2, GC), lambda i: (i, 0)),
                   pl.BlockSpec((_R2, H * GC), lambda i: (i, 0))],
        out_shape=[jax.ShapeDtypeStruct((N, GC), f32),
                   jax.ShapeDtypeStruct((N, H * GC), f32)],
        interpret=_INTERPRET,
    )(x1raw, stats1, xs, g1g, g1b, gat2_w)

    x2raw, stats2 = gat(h2, gat2_asrc, gat2_adst, g2bias)

    out = pl.pallas_call(
        _final_kernel,
        grid=(_G2,),
        in_specs=[pl.BlockSpec((_R2, GC), lambda i: (i, 0)), _full((8, GC)),
                  pl.BlockSpec((_R2, GC), lambda i: (i, 0)),
                  _full((1, GC)), _full((1, GC)),
                  _full(fc_w.shape), _full(fcb.shape)],
        out_specs=pl.BlockSpec((_BPG, NCLS), lambda i: (i, 0)),
        out_shape=jax.ShapeDtypeStruct((B, NCLS), f32),
        interpret=_INTERPRET,
    )(x2raw, stats2, x1f, g2g, g2b, fc_w, fcb)
    if _DBG:
        return dict(stats=stats, xc=xc, colsum=colsum, h1=h1, xs=xs,
                    x1raw=x1raw, stats1=stats1, x1f=x1f, h2=h2,
                    x2raw=x2raw, stats2=stats2, out=out)
    return out
